# Initial kernel scaffold; baseline (speedup 1.0000x reference)
import jax
import jax.numpy as jnp
from jax.experimental import pallas as pl


def kernel(node_features, rots, trans, edge_features, edge_index, seq_edge_features, seq_edge_index, res_mask, noising_mask, params):
    raise NotImplementedError("write your pallas kernel here")


# speedup vs baseline: 26.8951x; 26.8951x over previous
import functools
import math

import jax
import jax.numpy as jnp
import numpy as np
from jax import lax
from jax.experimental import pallas as pl
from jax.experimental.pallas import tpu as pltpu
from jax.experimental.pallas import tpu_sc as plsc

N = 10000
NP = N + 112
CH = 128
RPT = NP // 16
C_S = 128
C_Z = 64
C_H = 16
H = 8
PQK = 4
PV = 8

TBL_W = 272


def _ln_blk(x, g, b):
    mu = jnp.mean(x, -1, keepdims=True)
    var = jnp.mean((x - mu) ** 2, -1, keepdims=True)
    return (x - mu) * jax.lax.rsqrt(var + 1e-5) * g + b


def _tables_body(s_ref, r_ref, t_ref, wq_ref, bq_ref, wk_ref, bk_ref,
                 wv_ref, bv_ref, wqp_ref, bqp_ref, wkp_ref, bkp_ref,
                 wvp_ref, bvp_ref, P_ref, T_ref, S_ref, scq_ref,
                 qtab_ref, ktab_ref, vtab_ref, vptab0_ref, vptab1_ref,
                 mq_ref, mk_ref):
    s = s_ref[...]
    r = r_ref[...]
    t = t_ref[...]
    q = jnp.dot(s, wq_ref[...], preferred_element_type=jnp.float32,
                precision=jax.lax.Precision.HIGHEST) + bq_ref[...]
    k = jnp.dot(s, wk_ref[...], preferred_element_type=jnp.float32,
                precision=jax.lax.Precision.HIGHEST) + bk_ref[...]
    v = jnp.dot(s, wv_ref[...], preferred_element_type=jnp.float32,
                precision=jax.lax.Precision.HIGHEST) + bv_ref[...]

    def rot_pad(w_ref, b_ref, scale_vec):
        raw = jnp.dot(s, w_ref[...], preferred_element_type=jnp.float32,
                precision=jax.lax.Precision.HIGHEST) + b_ref[...]
        pad = jnp.dot(t, T_ref[...], preferred_element_type=jnp.float32,
                precision=jax.lax.Precision.HIGHEST)
        for i in range(3):
            rot_i = (r[:, 3 * i + 0:3 * i + 1] * raw[:, 0:32]
                     + r[:, 3 * i + 1:3 * i + 2] * raw[:, 32:64]
                     + r[:, 3 * i + 2:3 * i + 3] * raw[:, 64:96])
            pad += jnp.dot(rot_i, P_ref[i], preferred_element_type=jnp.float32,
                precision=jax.lax.Precision.HIGHEST)
        pad = pad * scale_vec
        n2 = jnp.dot(pad * pad, S_ref[...], preferred_element_type=jnp.float32,
                precision=jax.lax.Precision.HIGHEST)
        return pad, -0.5 * n2

    sc = scq_ref[...]
    qp_pad, qn2 = rot_pad(wqp_ref, bqp_ref, sc)
    kp_pad, kn2 = rot_pad(wkp_ref, bkp_ref, sc)
    c1 = 1.0 / math.sqrt(3.0 * C_H)
    qs = q * c1
    qtab_ref[:, 0:128] = qs
    qtab_ref[:, 128:256] = qp_pad
    qtab_ref[:, 256:272] = qn2
    ktab_ref[:, 0:128] = k
    ktab_ref[:, 128:256] = kp_pad
    ktab_ref[:, 256:272] = kn2
    vtab_ref[...] = v
    mq_ref[:, 0:16] = qs[:, 0:16] * (1.0 / 16.0)
    mq_ref[:, 16:32] = qp_pad[:, 0:16] * 0.25
    mq_ref[:, 32:48] = jnp.broadcast_to(qn2[:, 0:1] * (1.0 / 16.0),
                                        (qn2.shape[0], 16))
    mk_ref[:, 0:16] = k[:, 0:16]
    mk_ref[:, 16:32] = kp_pad[:, 0:16] * 0.25
    mk_ref[:, 32:48] = jnp.broadcast_to(kn2[:, 0:1] * (1.0 / 16.0),
                                        (kn2.shape[0], 16))
    wvp = wvp_ref[...]
    bvp = bvp_ref[...]
    raw = jnp.dot(s, wvp, preferred_element_type=jnp.float32,
                precision=jax.lax.Precision.HIGHEST) + bvp
    rot = []
    for i in range(3):
        rot.append(r[:, 3 * i + 0:3 * i + 1] * raw[:, 0:64]
                   + r[:, 3 * i + 1:3 * i + 2] * raw[:, 64:128]
                   + r[:, 3 * i + 2:3 * i + 3] * raw[:, 128:192]
                   + t[:, i:i + 1])
    vptab0_ref[:, 0:64] = rot[0]
    vptab0_ref[:, 64:96] = rot[1][:, 0:32]
    vptab1_ref[:, 0:32] = rot[1][:, 32:64]
    vptab1_ref[:, 32:96] = rot[2]


def _make_tables(s, r9, t, wq, bq, wk, bk, wv, bv, wqp, bqp, wkp, bkp,
                 wvp, bvp, Pmat, Tmat, Smat, scq):
    BN = 1264
    grid = (NP // BN,)
    full = lambda a: pl.BlockSpec(a.shape, lambda i: (0,) * a.ndim)
    row = lambda w: pl.BlockSpec((BN, w), lambda i: (i, 0))
    return pl.pallas_call(
        _tables_body,
        grid=grid,
        in_specs=[row(C_S), row(9), row(3)] + [full(a) for a in
                  (wq, bq, wk, bk, wv, bv, wqp, bqp, wkp, bkp, wvp, bvp,
                   Pmat, Tmat, Smat, scq)],
        out_specs=[row(TBL_W), row(TBL_W), row(128), row(96), row(96),
                   row(48), row(48)],
        out_shape=[jax.ShapeDtypeStruct((NP, TBL_W), jnp.float32),
                   jax.ShapeDtypeStruct((NP, TBL_W), jnp.float32),
                   jax.ShapeDtypeStruct((NP, 128), jnp.float32),
                   jax.ShapeDtypeStruct((NP, 96), jnp.float32),
                   jax.ShapeDtypeStruct((NP, 96), jnp.float32),
                   jax.ShapeDtypeStruct((NP, 48), jnp.float32),
                   jax.ShapeDtypeStruct((NP, 48), jnp.float32)],
    )(s, r9, t, wq, bq, wk, bk, wv, bv, wqp, bqp, wkp, bkp, wvp, bvp,
      Pmat, Tmat, Smat, scq)


def _bs_body(z_ref, w_ref, b_ref, out_ref):
    out_ref[...] = jnp.dot(z_ref[...], w_ref[...],
                           preferred_element_type=jnp.float32) + b_ref[...]


def _make_bs(z, wb, bb):
    E = z.shape[0]
    BE = 1024
    assert E % BE == 0
    return pl.pallas_call(
        _bs_body,
        grid=(E // BE,),
        in_specs=[pl.BlockSpec((BE, C_Z), lambda i: (i, 0)),
                  pl.BlockSpec(wb.shape, lambda i: (0, 0)),
                  pl.BlockSpec(bb.shape, lambda i: (0,))],
        out_specs=pl.BlockSpec((BE, H), lambda i: (i, 0)),
        out_shape=jax.ShapeDtypeStruct((E, H), jnp.float32),
    )(z, wb * math.sqrt(1.0 / 3.0), bb * math.sqrt(1.0 / 3.0))


def _fin_body(s_ref, r_ref, t_ref, accv_ref, accvp_ref, uop_ref,
              E8h_ref, Wo_ref, Woi_ref, Won_ref, bo_ref,
              g_ref, b_ref, out_ref):
    accv = accv_ref[0] + accv_ref[1]
    den8 = accv[:, 128:136]
    recip = jnp.where(den8 > 0.0, 1.0 / jnp.where(den8 > 0.0, den8, 1.0), 0.0)
    E8h = E8h_ref[...]
    recip64 = jnp.dot(recip, E8h, preferred_element_type=jnp.float32,
                precision=jax.lax.Precision.HIGHEST)
    ow = accv[:, 0:128]
    o = jnp.concatenate(
        [ow[:, 16 * h:16 * (h + 1)] * recip[:, h:h + 1] for h in range(H)], axis=1)
    accvp = accvp_ref[0] + accvp_ref[1]
    t = t_ref[...]
    r = r_ref[...]
    op = [accvp[:, 64 * i:64 * (i + 1)] * recip64 - t[:, i:i + 1]
          for i in range(3)]
    orot = [r[:, 0 + i:1 + i] * op[0] + r[:, 3 + i:4 + i] * op[1]
            + r[:, 6 + i:7 + i] * op[2] for i in range(3)]
    opn = jnp.sqrt(orot[0] ** 2 + orot[1] ** 2 + orot[2] ** 2 + 1e-8)
    u = (jnp.dot(o, Wo_ref[...], preferred_element_type=jnp.float32,
                precision=jax.lax.Precision.HIGHEST)
         + bo_ref[...]
         + jnp.dot(orot[0], Woi_ref[0], preferred_element_type=jnp.float32,
                precision=jax.lax.Precision.HIGHEST)
         + jnp.dot(orot[1], Woi_ref[1], preferred_element_type=jnp.float32,
                precision=jax.lax.Precision.HIGHEST)
         + jnp.dot(orot[2], Woi_ref[2], preferred_element_type=jnp.float32,
                precision=jax.lax.Precision.HIGHEST)
         + jnp.dot(opn, Won_ref[...], preferred_element_type=jnp.float32,
                precision=jax.lax.Precision.HIGHEST))
    u += uop_ref[0] + uop_ref[1]
    x = s_ref[...] + u
    out_ref[...] = _ln_blk(x, g_ref[...], b_ref[...])


def _make_finalize(s, r9, t, accv, accvp, uop, E8h, Wo, Woi, Won, bo,
                   ln_g, ln_b):
    BN = 400
    full = lambda a: pl.BlockSpec(a.shape, lambda i: (0,) * a.ndim)
    return pl.pallas_call(
        _fin_body,
        grid=(N // BN,),
        in_specs=[pl.BlockSpec((BN, C_S), lambda i: (i, 0)),
                  pl.BlockSpec((BN, 9), lambda i: (i, 0)),
                  pl.BlockSpec((BN, 3), lambda i: (i, 0)),
                  pl.BlockSpec((2, BN, 144), lambda i: (0, i, 0)),
                  pl.BlockSpec((2, BN, 192), lambda i: (0, i, 0)),
                  pl.BlockSpec((2, BN, 128), lambda i: (0, i, 0)),
                  full(E8h), full(Wo), full(Woi), full(Won),
                  full(bo), full(ln_g), full(ln_b)],
        out_specs=pl.BlockSpec((BN, C_S), lambda i: (i, 0)),
        out_shape=jax.ShapeDtypeStruct((N, C_S), jnp.float32),
    )(s, r9, t, accv, accvp, uop, E8h, Wo, Woi, Won, bo, ln_g, ln_b)


def _node_fin_body(s_ref, r_ref, t_ref, nm_ref, w1, b1, w2, b2, w3, b3,
                   g_ref, be_ref, wbb, bbb, s_out, rn_out, tn_out):
    s = s_ref[...]
    x = jnp.maximum(jnp.dot(s, w1[...], preferred_element_type=jnp.float32,
                precision=jax.lax.Precision.HIGHEST) + b1[...], 0.0)
    x = jnp.maximum(jnp.dot(x, w2[...], preferred_element_type=jnp.float32,
                precision=jax.lax.Precision.HIGHEST) + b2[...], 0.0)
    x = jnp.dot(x, w3[...], preferred_element_type=jnp.float32,
                precision=jax.lax.Precision.HIGHEST) + b3[...]
    s3 = _ln_blk(s + x, g_ref[...], be_ref[...])
    s_out[...] = s3
    nm = nm_ref[...]
    upd = (jnp.dot(s3 * nm, wbb[...], preferred_element_type=jnp.float32,
                precision=jax.lax.Precision.HIGHEST)
           + bbb[...]) * nm
    u0 = upd[:, 0:1]; u1 = upd[:, 1:2]; u2 = upd[:, 2:3]
    n2 = 1.0 + u0 * u0 + u1 * u1 + u2 * u2
    inv = 1.0 / n2
    r00 = 1.0 - 2.0 * (u1 * u1 + u2 * u2) * inv
    r01 = 2.0 * (u0 * u1 - u2) * inv
    r02 = 2.0 * (u0 * u2 + u1) * inv
    r10 = 2.0 * (u0 * u1 + u2) * inv
    r11 = 1.0 - 2.0 * (u0 * u0 + u2 * u2) * inv
    r12 = 2.0 * (u1 * u2 - u0) * inv
    r20 = 2.0 * (u0 * u2 - u1) * inv
    r21 = 2.0 * (u1 * u2 + u0) * inv
    r22 = 1.0 - 2.0 * (u0 * u0 + u1 * u1) * inv
    rq = [[r00, r01, r02], [r10, r11, r12], [r20, r21, r22]]
    r = r_ref[...]
    t = t_ref[...]
    for i in range(3):
        for kk in range(3):
            rn_out[:, 3 * i + kk:3 * i + kk + 1] = (
                r[:, 3 * i + 0:3 * i + 1] * rq[0][kk]
                + r[:, 3 * i + 1:3 * i + 2] * rq[1][kk]
                + r[:, 3 * i + 2:3 * i + 3] * rq[2][kk])
        tn_out[:, i:i + 1] = (t[:, i:i + 1]
                              + r[:, 3 * i + 0:3 * i + 1] * upd[:, 3:4]
                              + r[:, 3 * i + 1:3 * i + 2] * upd[:, 4:5]
                              + r[:, 3 * i + 2:3 * i + 3] * upd[:, 5:6])


def _make_node_fin(s, r9, t, nm, p_nt, p_bb):
    BN = 400
    full = lambda a: pl.BlockSpec(a.shape, lambda i: (0,) * a.ndim)
    args = (p_nt['l1']['w'], p_nt['l1']['b'], p_nt['l2']['w'], p_nt['l2']['b'],
            p_nt['l3']['w'], p_nt['l3']['b'], p_nt['ln']['g'], p_nt['ln']['b'],
            p_bb['w'], p_bb['b'])
    return pl.pallas_call(
        _node_fin_body,
        grid=(N // BN,),
        in_specs=[pl.BlockSpec((BN, C_S), lambda i: (i, 0)),
                  pl.BlockSpec((BN, 9), lambda i: (i, 0)),
                  pl.BlockSpec((BN, 3), lambda i: (i, 0)),
                  pl.BlockSpec((BN, 1), lambda i: (i, 0))] +
                 [full(a) for a in args],
        out_specs=[pl.BlockSpec((BN, C_S), lambda i: (i, 0)),
                   pl.BlockSpec((BN, 9), lambda i: (i, 0)),
                   pl.BlockSpec((BN, 3), lambda i: (i, 0))],
        out_shape=[jax.ShapeDtypeStruct((N, C_S), jnp.float32),
                   jax.ShapeDtypeStruct((N, 9), jnp.float32),
                   jax.ShapeDtypeStruct((N, 3), jnp.float32)],
    )(s, r9, t, nm, *args)


def _ne_body(s_ref, w1, b1, w2, b2, o1, o2):
    s = s_ref[...]
    o1[...] = jnp.dot(s, w1[...], preferred_element_type=jnp.float32,
                precision=jax.lax.Precision.HIGHEST) + b1[...]
    o2[...] = jnp.dot(s, w2[...], preferred_element_type=jnp.float32,
                precision=jax.lax.Precision.HIGHEST) + b2[...]


def _make_ne(s, p_et, p_set):
    BN = 400
    full = lambda a: pl.BlockSpec(a.shape, lambda i: (0,) * a.ndim)
    args = (p_et['init']['w'], p_et['init']['b'],
            p_set['init']['w'], p_set['init']['b'])
    return pl.pallas_call(
        _ne_body,
        grid=(N // BN,),
        in_specs=[pl.BlockSpec((BN, C_S), lambda i: (i, 0))] +
                 [full(a) for a in args],
        out_specs=[pl.BlockSpec((BN, 64), lambda i: (i, 0)),
                   pl.BlockSpec((BN, 64), lambda i: (i, 0))],
        out_shape=[jax.ShapeDtypeStruct((N, 64), jnp.float32),
                   jax.ShapeDtypeStruct((N, 64), jnp.float32)],
    )(s, *args)


def _et_body(z_ref, ns_ref, nd_ref, w1a, w1b, w1c, b1, w2, b2,
             wfa, wfb, wfc, wfx, bf, g_ref, be_ref, out_ref):
    z = z_ref[...]
    ns = ns_ref[...]
    nd = nd_ref[...]
    x1 = jnp.maximum(
        jnp.dot(z, w1a[...], preferred_element_type=jnp.float32)
        + jnp.dot(ns, w1b[...], preferred_element_type=jnp.float32)
        + jnp.dot(nd, w1c[...], preferred_element_type=jnp.float32)
        + b1[...], 0.0)
    x2 = jnp.maximum(
        jnp.dot(x1, w2[...], preferred_element_type=jnp.float32) + b2[...], 0.0)
    e = (jnp.dot(x2, wfx[...], preferred_element_type=jnp.float32)
         + jnp.dot(z, wfa[...], preferred_element_type=jnp.float32)
         + jnp.dot(ns, wfb[...], preferred_element_type=jnp.float32)
         + jnp.dot(nd, wfc[...], preferred_element_type=jnp.float32)
         + bf[...])
    out_ref[...] = _ln_blk(e, g_ref[...], be_ref[...])


def _make_et(z, ns, nd, p):
    E = z.shape[0]
    BE = 1024
    assert E % BE == 0
    w1 = p['t1']['w']
    wf = p['final']['w']
    args = (w1[0:64], w1[64:128], w1[128:192], p['t1']['b'],
            p['t2']['w'], p['t2']['b'],
            wf[0:64], wf[64:128], wf[128:192], wf,
            p['final']['b'], p['ln']['g'], p['ln']['b'])
    full = lambda a: pl.BlockSpec(a.shape, lambda i: (0,) * a.ndim)
    return pl.pallas_call(
        _et_body,
        grid=(E // BE,),
        in_specs=[pl.BlockSpec((BE, 64), lambda i: (i, 0))] * 3 +
                 [full(a) for a in args],
        out_specs=pl.BlockSpec((BE, 64), lambda i: (i, 0)),
        out_shape=jax.ShapeDtypeStruct((E, 64), jnp.float32),
    )(z, ns, nd, *args)


def _sc_mesh():
    return plsc.VectorSubcoreMesh(core_axis_name="c", subcore_axis_name="s")


def _allsum(v, lane):
    for sh in (1, 2, 4, 8):
        v = v + jnp.take(v, lane ^ sh)
    return v


def _zero_acc(rows_ref, acc_ref, sid, width):
    zv = jnp.zeros((16,), jnp.float32)
    nr = rows_ref.shape[0]

    def zr(r, _):
        for wb in range(width // 16):
            rows_ref[r, pl.ds(wb * 16, 16)] = zv
        return 0
    lax.fori_loop(0, nr, zr, 0)
    r0 = sid * RPT
    nfull = RPT // nr

    def zc(i, _):
        pltpu.sync_copy(rows_ref, acc_ref.at[pl.ds(r0 + i * nr, nr)])
        return 0
    lax.fori_loop(0, nfull, zc, 0)
    rem = RPT - nfull * nr
    if rem:
        pltpu.sync_copy(rows_ref.at[pl.ds(0, rem)],
                        acc_ref.at[pl.ds(r0 + nfull * nr, rem)])


def _edge_split(Epad):
    Epc = Epad // 2
    nch = Epc // CH
    kmax = (nch + 15) // 16
    return Epc, nch, kmax


def _sc_pass_a(mq, mk, src, dst):
    Epad = src.shape[0]
    Epc, nch, kmax = _edge_split(Epad)

    @functools.partial(
        pl.kernel,
        out_type=jax.ShapeDtypeStruct((2, NP, 16), jnp.float32),
        mesh=_sc_mesh(),
        compiler_params=pltpu.CompilerParams(use_tc_tiling_on_sc=False),
        scratch_types=[
            pltpu.VMEM((CH,), jnp.int32), pltpu.VMEM((CH,), jnp.int32),
            pltpu.VMEM((CH, 48), jnp.float32), pltpu.VMEM((CH, 48), jnp.float32),
            pltpu.VMEM((CH,), jnp.float32), pltpu.VMEM((CH, 16), jnp.float32),
            pltpu.SemaphoreType.DMA, pltpu.SemaphoreType.DMA,
            pltpu.VMEM_SHARED((NP, 16), jnp.float32),
        ],
    )
    def kern(mq_h, mk_h, src_h, dst_h, out_h, sidx, didx, mqr, mkr, abuf,
             rows, sem1, sem2, acc):
        cid = lax.axis_index("c")
        sid = lax.axis_index("s")
        lane = lax.iota(jnp.int32, 16)
        zcol = jnp.zeros((16,), jnp.int32)
        _zero_acc(rows, acc, sid, 16)
        plsc.subcore_barrier()

        def chunk(k, _):
            ch = sid + k * 16

            @pl.when(ch < nch)
            def _():
                e0 = cid * Epc + ch * CH
                pltpu.sync_copy(src_h.at[pl.ds(e0, CH)], sidx)
                pltpu.sync_copy(dst_h.at[pl.ds(e0, CH)], didx)
                c1 = pltpu.async_copy(mq_h.at[didx], mqr, sem1)
                c2 = pltpu.async_copy(mk_h.at[sidx], mkr, sem2)
                c1.wait()
                c2.wait()

                def edge(r, _):
                    d = (mqr[r, pl.ds(0, 16)] * mkr[r, pl.ds(0, 16)]
                         + mqr[r, pl.ds(16, 16)] * mkr[r, pl.ds(16, 16)])
                    a0 = (_allsum(d, lane) + mqr[r, pl.ds(32, 16)]
                          + mkr[r, pl.ds(32, 16)])
                    rows[r, :] = jnp.exp(a0)
                    return 0
                lax.fori_loop(0, CH, edge, 0)
                pltpu.sync_copy(rows, acc.at[didx], add=True)
            return 0
        lax.fori_loop(0, kmax, chunk, 0)
        plsc.subcore_barrier()
        pltpu.sync_copy(acc.at[pl.ds(sid * RPT, RPT)],
                        out_h.at[cid, pl.ds(sid * RPT, RPT)])
    return kern(mq, mk, src, dst)


def _mprep_body(u_ref, out_ref):
    u = u_ref[0, :, 0:1] + u_ref[1, :, 0:1]
    m = jnp.where(u > 1e-35, 16.0 * jnp.log(jnp.maximum(u, 1e-35)), 0.0)
    out_ref[...] = jnp.broadcast_to(m, (u.shape[0], 16))


def _make_mtab(U):
    BN = 1264
    return pl.pallas_call(
        _mprep_body,
        grid=(NP // BN,),
        in_specs=[pl.BlockSpec((2, BN, 16), lambda i: (0, i, 0))],
        out_specs=pl.BlockSpec((BN, 16), lambda i: (i, 0)),
        out_shape=jax.ShapeDtypeStruct((NP, 16), jnp.float32),
    )(U)


def _sc_scores(qtab, ktab, mtab, bsf, src, dst):
    Epad = src.shape[0]
    C = CH // 2
    Epc = Epad // 2
    nch = Epc // C
    kmax = (nch + 15) // 16

    @functools.partial(
        pl.kernel,
        out_type=jax.ShapeDtypeStruct((Epad * 8,), jnp.float32),
        mesh=_sc_mesh(),
        compiler_params=pltpu.CompilerParams(use_tc_tiling_on_sc=False),
        scratch_types=[
            pltpu.VMEM((C,), jnp.int32), pltpu.VMEM((C,), jnp.int32),
            pltpu.VMEM((C, TBL_W), jnp.float32),
            pltpu.VMEM((C, TBL_W), jnp.float32),
            pltpu.VMEM((C, 16), jnp.float32),
            pltpu.VMEM((C * 8,), jnp.float32),
            pltpu.SemaphoreType.DMA, pltpu.SemaphoreType.DMA,
            pltpu.SemaphoreType.DMA,
            pltpu.VMEM((C,), jnp.int32), pltpu.VMEM((C,), jnp.int32),
            pltpu.VMEM((C, TBL_W), jnp.float32),
            pltpu.VMEM((C, TBL_W), jnp.float32),
            pltpu.VMEM((C, 16), jnp.float32),
            pltpu.VMEM((C * 8,), jnp.float32),
            pltpu.SemaphoreType.DMA, pltpu.SemaphoreType.DMA,
            pltpu.SemaphoreType.DMA,
            pltpu.VMEM((C * 8,), jnp.float32),
        ],
    )
    def kern(q_h, k_h, m_h, bs_h, src_h, dst_h, ea_h,
             sidx0, didx0, qr0, kr0, mr0, bsb0, semq0, semk0, semm0,
             sidx1, didx1, qr1, kr1, mr1, bsb1, semq1, semk1, semm1, ab):
        cid = lax.axis_index("c")
        sid = lax.axis_index("s")
        lane = lax.iota(jnp.int32, 16)
        msk = lane < 8
        bufs = ((sidx0, didx0, qr0, kr0, mr0, bsb0, semq0, semk0, semm0),
                (sidx1, didx1, qr1, kr1, mr1, bsb1, semq1, semk1, semm1))

        def issue(ch, b):
            sidx, didx, qr, kr, mr, bsb, semq, semk, semm = b
            e0 = cid * Epc + ch * C
            pltpu.sync_copy(src_h.at[pl.ds(e0, C)], sidx)
            pltpu.sync_copy(dst_h.at[pl.ds(e0, C)], didx)
            pltpu.sync_copy(bs_h.at[pl.ds(e0 * 8, C * 8)], bsb)
            pltpu.async_copy(q_h.at[didx], qr, semq)
            pltpu.async_copy(k_h.at[sidx], kr, semk)
            pltpu.async_copy(m_h.at[didx], mr, semm)

        def work(ch, b):
            sidx, didx, qr, kr, mr, bsb, semq, semk, semm = b
            e0 = cid * Epc + ch * C
            pltpu.make_async_copy(q_h.at[didx], qr, semq).wait()
            pltpu.make_async_copy(k_h.at[sidx], kr, semk).wait()
            pltpu.make_async_copy(m_h.at[didx], mr, semm).wait()

            def pair(u, _):
                av = jnp.zeros((16,), jnp.float32)
                for l in range(16):
                    rr = 2 * u + (l // 8)
                    off = (l % 8) * 16
                    d = (qr[rr, pl.ds(off, 16)] * kr[rr, pl.ds(off, 16)]
                         + qr[rr, pl.ds(off + 128, 16)]
                         * kr[rr, pl.ds(off + 128, 16)])
                    av = jnp.where(lane == l, _allsum(d, lane), av)
                nq = jnp.where(msk, qr[2 * u, pl.ds(256, 16)],
                               qr[2 * u + 1, pl.ds(256, 16)])
                nk = jnp.where(msk, kr[2 * u, pl.ds(256, 16)],
                               kr[2 * u + 1, pl.ds(256, 16)])
                mv = jnp.where(msk, mr[2 * u, :], mr[2 * u + 1, :])
                ab[pl.ds(u * 16, 16)] = jnp.exp(
                    av + nq + nk - mv + bsb[pl.ds(u * 16, 16)])
                return 0
            lax.fori_loop(0, C // 2, pair, 0)
            pltpu.sync_copy(ab, ea_h.at[pl.ds(e0 * 8, C * 8)])

        @pl.when(sid < nch)
        def _():
            issue(sid, bufs[0])

        def two(j, _):
            for par in (0, 1):
                k = 2 * j + par
                ch = sid + k * 16

                @pl.when(ch < nch)
                def _(ch=ch, par=par):
                    @pl.when(ch + 16 < nch)
                    def _():
                        issue(ch + 16, bufs[1 - par])
                    work(ch, bufs[par])
            return 0
        lax.fori_loop(0, (kmax + 1) // 2, two, 0)
    return kern(qtab, ktab, mtab, bsf, src, dst)


def _sc_accumulate(tab_or_z, eaf, src, dst, mode, h0=0, hp=()):
    Epad = src.shape[0]
    C = 64 if mode == 'v' else CH
    Epc = Epad // 2
    nch = Epc // C
    kmax = (nch + 15) // 16
    width = {'v': 144, 'vp': 96, 'z': 128}[mode]
    in_w = {'v': 128, 'vp': 96, 'z': 64}[mode]

    @functools.partial(
        pl.kernel,
        out_type=jax.ShapeDtypeStruct((2, NP, width), jnp.float32),
        mesh=_sc_mesh(),
        compiler_params=pltpu.CompilerParams(use_tc_tiling_on_sc=False),
        scratch_types=[
            pltpu.VMEM((C,), jnp.int32), pltpu.VMEM((C,), jnp.int32),
            pltpu.VMEM((C, in_w), jnp.float32),
            pltpu.VMEM((C * 8 + 16,), jnp.float32),
            pltpu.SemaphoreType.DMA,
            pltpu.VMEM((C,), jnp.int32), pltpu.VMEM((C,), jnp.int32),
            pltpu.VMEM((C, in_w), jnp.float32),
            pltpu.VMEM((C * 8 + 16,), jnp.float32),
            pltpu.SemaphoreType.DMA,
            pltpu.VMEM((C, width), jnp.float32),
            pltpu.VMEM_SHARED((NP, width), jnp.float32),
        ],
    )
    def kern(t_h, ea_h, src_h, dst_h, out_h, sidx0, didx0, tr0, eab0, sem0,
             sidx1, didx1, tr1, eab1, sem1, rows, acc):
        cid = lax.axis_index("c")
        sid = lax.axis_index("s")
        msk = lax.iota(jnp.int32, 16) < 8
        _zero_acc(rows, acc, sid, width)
        eab0[pl.ds(C * 8, 16)] = jnp.zeros((16,), jnp.float32)
        eab1[pl.ds(C * 8, 16)] = jnp.zeros((16,), jnp.float32)
        plsc.subcore_barrier()
        bufs = ((sidx0, didx0, tr0, eab0, sem0),
                (sidx1, didx1, tr1, eab1, sem1))

        def issue(ch, b):
            sidx, didx, tr, eab, sem = b
            e0 = cid * Epc + ch * C
            pltpu.sync_copy(src_h.at[pl.ds(e0, C)], sidx)
            pltpu.sync_copy(dst_h.at[pl.ds(e0, C)], didx)
            pltpu.sync_copy(ea_h.at[pl.ds(e0 * 8, C * 8)],
                            eab.at[pl.ds(0, C * 8)])
            if mode == 'z':
                pltpu.async_copy(t_h.at[pl.ds(e0, C)], tr, sem)
            else:
                pltpu.async_copy(t_h.at[sidx], tr, sem)

        def work(b):
            sidx, didx, tr, eab, sem = b
            if mode == 'z':
                pltpu.make_async_copy(t_h.at[pl.ds(0, C)], tr, sem).wait()
            else:
                pltpu.make_async_copy(t_h.at[sidx], tr, sem).wait()

            def edge(r, _):
                ev = eab[pl.ds(r * 8, 16)]
                if mode == 'v':
                    rows[r, pl.ds(128, 16)] = ev
                    for h in range(8):
                        w = jnp.full((16,), ev[h], jnp.float32)
                        rows[r, pl.ds(h * 16, 16)] = (
                            w * tr[r, pl.ds(h * 16, 16)])
                elif mode == 'vp':
                    for cb in range(6):
                        h = hp[cb]
                        w = jnp.where(
                            msk,
                            jnp.full((16,), ev[h], jnp.float32),
                            jnp.full((16,), ev[h + 1], jnp.float32))
                        rows[r, pl.ds(cb * 16, 16)] = (
                            w * tr[r, pl.ds(cb * 16, 16)])
                else:
                    for j in range(2):
                        w = jnp.full((16,), ev[h0 + j], jnp.float32)
                        for cb in range(4):
                            rows[r, pl.ds(j * 64 + cb * 16, 16)] = (
                                w * tr[r, pl.ds(cb * 16, 16)])
                return 0
            lax.fori_loop(0, C, edge, 0)
            pltpu.sync_copy(rows, acc.at[didx], add=True)

        @pl.when(sid < nch)
        def _():
            issue(sid, bufs[0])

        def two(j, _):
            for par in (0, 1):
                k = 2 * j + par
                ch = sid + k * 16

                @pl.when(ch < nch)
                def _(ch=ch, par=par):
                    @pl.when(ch + 16 < nch)
                    def _():
                        issue(ch + 16, bufs[1 - par])
                    work(bufs[par])
            return 0
        lax.fori_loop(0, (kmax + 1) // 2, two, 0)
        plsc.subcore_barrier()
        pltpu.sync_copy(acc.at[pl.ds(sid * RPT, RPT)],
                        out_h.at[cid, pl.ds(sid * RPT, RPT)])
    return kern(tab_or_z, eaf, src, dst)


def _dentab_body(accv_ref, out_ref):
    den8 = accv_ref[0][:, 128:136] + accv_ref[1][:, 128:136]
    inv = jnp.where(den8 > 0.0, 1.0 / jnp.where(den8 > 0.0, den8, 1.0), 0.0)
    out_ref[...] = jnp.concatenate([inv, jnp.zeros_like(inv)], axis=1)


def _make_dentab(accv):
    BN = 1264
    return pl.pallas_call(
        _dentab_body,
        grid=(NP // BN,),
        in_specs=[pl.BlockSpec((2, BN, 144), lambda i: (0, i, 0))],
        out_specs=pl.BlockSpec((BN, 16), lambda i: (i, 0)),
        out_shape=jax.ShapeDtypeStruct((NP, 16), jnp.float32),
    )(accv)


def _yz_body(z_ref, ea_ref, inv_ref, wog_ref, out_ref):
    z = z_ref[...]
    attn = ea_ref[...] * inv_ref[:, 0:8]
    wog = wog_ref[...]
    acc = attn[:, 0:1] * jnp.dot(z, wog[0:64],
                                 preferred_element_type=jnp.float32)
    for h in range(1, H):
        acc += attn[:, h:h + 1] * jnp.dot(z, wog[64 * h:64 * (h + 1)],
                                          preferred_element_type=jnp.float32)
    out_ref[...] = acc


def _make_yz(z, ea, invd, Wog):
    E = z.shape[0]
    BE = 1024
    full = lambda a: pl.BlockSpec(a.shape, lambda i: (0,) * a.ndim)
    return pl.pallas_call(
        _yz_body,
        grid=(E // BE,),
        in_specs=[pl.BlockSpec((BE, C_Z), lambda i: (i, 0)),
                  pl.BlockSpec((BE, H), lambda i: (i, 0)),
                  pl.BlockSpec((BE, 16), lambda i: (i, 0)),
                  full(Wog)],
        out_specs=pl.BlockSpec((BE, 128), lambda i: (i, 0)),
        out_shape=jax.ShapeDtypeStruct((E, 128), jnp.float32),
    )(z, ea, invd, Wog)


def _sc_gather1(tab, idx):
    Epad = idx.shape[0]
    Epc, nch, kmax = _edge_split(Epad)
    w = tab.shape[1]

    @functools.partial(
        pl.kernel,
        out_type=jax.ShapeDtypeStruct((Epad, w), jnp.float32),
        mesh=_sc_mesh(),
        compiler_params=pltpu.CompilerParams(use_tc_tiling_on_sc=False),
        scratch_types=[
            pltpu.VMEM((CH,), jnp.int32),
            pltpu.VMEM((CH, w), jnp.float32),
            pltpu.SemaphoreType.DMA,
        ],
    )
    def kern(tab_h, idx_h, out_h, didx, buf, sem1):
        cid = lax.axis_index("c")
        sid = lax.axis_index("s")

        def chunk(k, _):
            ch = sid + k * 16

            @pl.when(ch < nch)
            def _():
                e0 = cid * Epc + ch * CH
                pltpu.sync_copy(idx_h.at[pl.ds(e0, CH)], didx)
                pltpu.async_copy(tab_h.at[didx], buf, sem1).wait()
                pltpu.sync_copy(buf, out_h.at[pl.ds(e0, CH)])
            return 0
        lax.fori_loop(0, kmax, chunk, 0)
    return kern(tab, idx)


def _sc_scatter_rows(rows_tab, src, dst):
    Epad = src.shape[0]
    Epc, nch, kmax = _edge_split(Epad)

    @functools.partial(
        pl.kernel,
        out_type=jax.ShapeDtypeStruct((2, NP, 128), jnp.float32),
        mesh=_sc_mesh(),
        compiler_params=pltpu.CompilerParams(use_tc_tiling_on_sc=False),
        scratch_types=[
            pltpu.VMEM((CH,), jnp.int32),
            pltpu.VMEM((CH, 128), jnp.float32),
            pltpu.SemaphoreType.DMA,
            pltpu.VMEM_SHARED((NP, 128), jnp.float32),
        ],
    )
    def kern(rows_h, dst_h, out_h, didx, buf, sem1, acc):
        cid = lax.axis_index("c")
        sid = lax.axis_index("s")
        _zero_acc(buf, acc, sid, 128)
        plsc.subcore_barrier()

        def chunk(k, _):
            ch = sid + k * 16

            @pl.when(ch < nch)
            def _():
                e0 = cid * Epc + ch * CH
                pltpu.sync_copy(dst_h.at[pl.ds(e0, CH)], didx)
                pltpu.async_copy(rows_h.at[pl.ds(e0, CH)], buf, sem1).wait()
                pltpu.sync_copy(buf, acc.at[didx], add=True)
            return 0
        lax.fori_loop(0, kmax, chunk, 0)
        plsc.subcore_barrier()
        pltpu.sync_copy(acc.at[pl.ds(sid * RPT, RPT)],
                        out_h.at[cid, pl.ds(sid * RPT, RPT)])
    return kern(rows_tab, dst)


def _sc_gather_ne(tab, src, dst):
    Epad = src.shape[0]
    Epc, nch, kmax = _edge_split(Epad)

    @functools.partial(
        pl.kernel,
        out_type=[jax.ShapeDtypeStruct((Epad, 64), jnp.float32),
                  jax.ShapeDtypeStruct((Epad, 64), jnp.float32)],
        mesh=_sc_mesh(),
        compiler_params=pltpu.CompilerParams(use_tc_tiling_on_sc=False),
        scratch_types=[
            pltpu.VMEM((CH,), jnp.int32), pltpu.VMEM((CH,), jnp.int32),
            pltpu.VMEM((CH, 64), jnp.float32), pltpu.VMEM((CH, 64), jnp.float32),
            pltpu.SemaphoreType.DMA, pltpu.SemaphoreType.DMA,
        ],
    )
    def kern(tab_h, src_h, dst_h, outs_h, outd_h, sidx, didx, b1, b2,
             sem1, sem2):
        cid = lax.axis_index("c")
        sid = lax.axis_index("s")

        def chunk(k, _):
            ch = sid + k * 16

            @pl.when(ch < nch)
            def _():
                e0 = cid * Epc + ch * CH
                pltpu.sync_copy(src_h.at[pl.ds(e0, CH)], sidx)
                pltpu.sync_copy(dst_h.at[pl.ds(e0, CH)], didx)
                c1 = pltpu.async_copy(tab_h.at[sidx], b1, sem1)
                c2 = pltpu.async_copy(tab_h.at[didx], b2, sem2)
                c1.wait()
                c2.wait()
                pltpu.sync_copy(b1, outs_h.at[pl.ds(e0, CH)])
                pltpu.sync_copy(b2, outd_h.at[pl.ds(e0, CH)])
            return 0
        lax.fori_loop(0, kmax, chunk, 0)
    return kern(tab, src, dst)


def _perm_ipa_weights(p):
    wq, bq = p['wq']['w'], p['wq']['b']
    wkv, bkv = p['wkv']['w'], p['wkv']['b']
    kcols = np.array([h * 32 + c for h in range(H) for c in range(C_H)])
    vcols = kcols + C_H
    wk, bk = wkv[:, kcols], bkv[kcols]
    wv, bv = wkv[:, vcols], bkv[vcols]
    qp_cols = np.array([hp * 3 + j for j in range(3) for hp in range(H * PQK)])
    wqp, bqp = p['wqp']['w'][:, qp_cols], p['wqp']['b'][qp_cols]
    kp_cols = np.array([(h * 12 + pq) * 3 + j
                        for j in range(3) for h in range(H) for pq in range(PQK)])
    vp_cols = np.array([(h * 12 + 4 + pv) * 3 + j
                        for j in range(3) for h in range(H) for pv in range(PV)])
    wkp, bkp = p['wkvp']['w'][:, kp_cols], p['wkvp']['b'][kp_cols]
    wvp, bvp = p['wkvp']['w'][:, vp_cols], p['wkvp']['b'][vp_cols]
    Pm = np.zeros((3, 32, 128), np.float32)
    Tm = np.zeros((3, 128), np.float32)
    for h in range(H):
        for i in range(3):
            for pq in range(PQK):
                Pm[i, h * 4 + pq, h * 16 + i * 4 + pq] = 1.0
                Tm[i, h * 16 + i * 4 + pq] = 1.0
    Sm = np.zeros((128, 16), np.float32)
    for h in range(H):
        Sm[h * 16:(h + 1) * 16, h] = 1.0
        Sm[h * 16:(h + 1) * 16, 8 + h] = 1.0
    hw2 = 0.5 * jax.nn.softplus(p['hw']) * math.sqrt(1.0 / (3.0 * (PQK * 9.0 / 2.0)))
    scq = jnp.sqrt(2.0 * hw2)[:, None].repeat(16, 1).reshape(1, 128)
    wo, bo = p['wo']['w'], p['wo']['b']
    Wo = wo[0:128]
    Woi = jnp.stack([wo[128 + np.array([h * 24 + pv * 3 + i
                                        for h in range(H) for pv in range(PV)])]
                     for i in range(3)], 0)
    Won = wo[320:384]
    Wog = wo[384:896]
    E8h = np.zeros((8, 64), np.float32)
    for h in range(H):
        E8h[h, h * 8:(h + 1) * 8] = 1.0
    return dict(wq=wq, bq=bq, wk=wk, bk=bk, wv=wv, bv=bv, wqp=wqp, bqp=bqp,
                wkp=wkp, bkp=bkp, wvp=wvp, bvp=bvp,
                P=jnp.asarray(Pm), T=jnp.asarray(Tm), S=jnp.asarray(Sm),
                scq=scq, Wo=Wo, Woi=Woi, Won=Won, Wog=Wog, bo=bo,
                E8h=jnp.asarray(E8h))


def _pad_rows(a, rows):
    return jnp.pad(a, ((0, rows - a.shape[0]),) + ((0, 0),) * (a.ndim - 1))


def _run_ipa(s, sp, r9, r9p, t, tp, zpad, bsf, src, dst, p, ln):
    w = _perm_ipa_weights(p)
    qtab, ktab, vtab, vptab0, vptab1, mq, mk = _make_tables(
        sp, r9p, tp, w['wq'], w['bq'], w['wk'], w['bk'], w['wv'], w['bv'],
        w['wqp'], w['bqp'], w['wkp'], w['bkp'], w['wvp'], w['bvp'],
        w['P'], w['T'], w['S'], w['scq'])
    U = _sc_pass_a(mq, mk, src, dst)
    mtab = _make_mtab(U)
    eaf = _sc_scores(qtab, ktab, mtab, bsf, src, dst)
    accv = _sc_accumulate(vtab, eaf, src, dst, 'v')
    accvp = jnp.concatenate(
        [_sc_accumulate(vptab0, eaf, src, dst, 'vp', hp=(0, 2, 4, 6, 0, 2)),
         _sc_accumulate(vptab1, eaf, src, dst, 'vp', hp=(4, 6, 0, 2, 4, 6))],
        axis=-1)
    dentab = _make_dentab(accv)
    invd = _sc_gather1(dentab, dst)
    yz = _make_yz(zpad, eaf.reshape(-1, 8), invd, w['Wog'])
    uop = _sc_scatter_rows(yz, src, dst)
    return _make_finalize(s, r9, t, accv, accvp, uop, w['E8h'], w['Wo'],
                          w['Woi'], w['Won'], w['bo'],
                          ln['g'], ln['b'])


def kernel(node_features, rots, trans, edge_features, edge_index,
           seq_edge_features, seq_edge_index, res_mask, noising_mask, params):
    r9 = rots.reshape(N, 9)
    t = trans
    r9p = _pad_rows(r9, NP)
    tp = _pad_rows(t, NP)

    def prep(z, ei):
        E0 = ei.shape[1]
        Epad = -(-E0 // 1024) * 1024
        srcp = jnp.pad(ei[0], (0, Epad - E0))
        dstp = jnp.pad(ei[1], (0, Epad - E0), constant_values=N)
        return jnp.pad(z, ((0, Epad - E0), (0, 0))), srcp, dstp, E0

    zp, src, dst, E = prep(edge_features, edge_index)
    szp, ssrc, sdst, ES = prep(seq_edge_features, seq_edge_index)
    bsf = _make_bs(zp, params['ipa_sp']['wb']['w'],
                   params['ipa_sp']['wb']['b']).reshape(-1)
    sbsf = _make_bs(szp, params['ipa_sq']['wb']['w'],
                    params['ipa_sq']['wb']['b']).reshape(-1)
    s0p = _pad_rows(node_features, NP)
    s1 = _run_ipa(node_features, s0p, r9, r9p, t, tp, zp, bsf, src, dst,
                  params['ipa_sp'], params['ln1'])
    s2 = _run_ipa(s1, _pad_rows(s1, NP), r9, r9p, t, tp, szp, sbsf,
                  ssrc, sdst, params['ipa_sq'], params['ln2'])
    s3, rn9, tn = _make_node_fin(s2, r9, t, noising_mask[:, None],
                                 params['nt'], params['bb'])
    ne_et, ne_set = _make_ne(s3, params['et'], params['set'])
    nsrc, ndst = _sc_gather_ne(_pad_rows(ne_et, NP), src, dst)
    e = _make_et(zp, nsrc, ndst, params['et'])[:E]
    snsrc, sndst = _sc_gather_ne(_pad_rows(ne_set, NP), ssrc, sdst)
    se = _make_et(szp, snsrc, sndst, params['set'])[:ES]
    return s3, rn9.reshape(N, 3, 3), tn, e, se


# speedup vs baseline: 27.2917x; 1.0147x over previous
import functools
import math

import jax
import jax.numpy as jnp
import numpy as np
from jax import lax
from jax.experimental import pallas as pl
from jax.experimental.pallas import tpu as pltpu
from jax.experimental.pallas import tpu_sc as plsc

N = 10000
NP = N + 112
CH = 128
RPT = NP // 16
C_S = 128
C_Z = 64
C_H = 16
H = 8
PQK = 4
PV = 8

TBL_W = 272


def _ln_blk(x, g, b):
    mu = jnp.mean(x, -1, keepdims=True)
    var = jnp.mean((x - mu) ** 2, -1, keepdims=True)
    return (x - mu) * jax.lax.rsqrt(var + 1e-5) * g + b


def _tables_body(s_ref, r_ref, t_ref, wq_ref, bq_ref, wk_ref, bk_ref,
                 wv_ref, bv_ref, wqp_ref, bqp_ref, wkp_ref, bkp_ref,
                 wvp_ref, bvp_ref, P_ref, T_ref, S_ref, scq_ref,
                 qtab_ref, ktab_ref, vtab_ref, vptab0_ref, vptab1_ref,
                 mq_ref, mk_ref):
    s = s_ref[...]
    r = r_ref[...]
    t = t_ref[...]
    q = jnp.dot(s, wq_ref[...], preferred_element_type=jnp.float32,
                precision=jax.lax.Precision.HIGHEST) + bq_ref[...]
    k = jnp.dot(s, wk_ref[...], preferred_element_type=jnp.float32,
                precision=jax.lax.Precision.HIGHEST) + bk_ref[...]
    v = jnp.dot(s, wv_ref[...], preferred_element_type=jnp.float32,
                precision=jax.lax.Precision.HIGHEST) + bv_ref[...]

    def rot_pad(w_ref, b_ref, scale_vec):
        raw = jnp.dot(s, w_ref[...], preferred_element_type=jnp.float32,
                precision=jax.lax.Precision.HIGHEST) + b_ref[...]
        pad = jnp.dot(t, T_ref[...], preferred_element_type=jnp.float32,
                precision=jax.lax.Precision.HIGHEST)
        for i in range(3):
            rot_i = (r[:, 3 * i + 0:3 * i + 1] * raw[:, 0:32]
                     + r[:, 3 * i + 1:3 * i + 2] * raw[:, 32:64]
                     + r[:, 3 * i + 2:3 * i + 3] * raw[:, 64:96])
            pad += jnp.dot(rot_i, P_ref[i], preferred_element_type=jnp.float32,
                precision=jax.lax.Precision.HIGHEST)
        pad = pad * scale_vec
        n2 = jnp.dot(pad * pad, S_ref[...], preferred_element_type=jnp.float32,
                precision=jax.lax.Precision.HIGHEST)
        return pad, -0.5 * n2

    sc = scq_ref[...]
    qp_pad, qn2 = rot_pad(wqp_ref, bqp_ref, sc)
    kp_pad, kn2 = rot_pad(wkp_ref, bkp_ref, sc)
    c1 = 1.0 / math.sqrt(3.0 * C_H)
    qs = q * c1
    qtab_ref[:, 0:128] = qs
    qtab_ref[:, 128:256] = qp_pad
    qtab_ref[:, 256:272] = qn2
    ktab_ref[:, 0:128] = k
    ktab_ref[:, 128:256] = kp_pad
    ktab_ref[:, 256:272] = kn2
    vtab_ref[...] = v
    mq_ref[:, 0:16] = qs[:, 0:16] * (1.0 / 16.0)
    mq_ref[:, 16:32] = qp_pad[:, 0:16] * 0.25
    mq_ref[:, 32:48] = jnp.broadcast_to(qn2[:, 0:1] * (1.0 / 16.0),
                                        (qn2.shape[0], 16))
    mk_ref[:, 0:16] = k[:, 0:16]
    mk_ref[:, 16:32] = kp_pad[:, 0:16] * 0.25
    mk_ref[:, 32:48] = jnp.broadcast_to(kn2[:, 0:1] * (1.0 / 16.0),
                                        (kn2.shape[0], 16))
    wvp = wvp_ref[...]
    bvp = bvp_ref[...]
    raw = jnp.dot(s, wvp, preferred_element_type=jnp.float32,
                precision=jax.lax.Precision.HIGHEST) + bvp
    rot = []
    for i in range(3):
        rot.append(r[:, 3 * i + 0:3 * i + 1] * raw[:, 0:64]
                   + r[:, 3 * i + 1:3 * i + 2] * raw[:, 64:128]
                   + r[:, 3 * i + 2:3 * i + 3] * raw[:, 128:192]
                   + t[:, i:i + 1])
    vptab0_ref[:, 0:64] = rot[0]
    vptab0_ref[:, 64:96] = rot[1][:, 0:32]
    vptab1_ref[:, 0:32] = rot[1][:, 32:64]
    vptab1_ref[:, 32:96] = rot[2]


def _make_tables(s, r9, t, wq, bq, wk, bk, wv, bv, wqp, bqp, wkp, bkp,
                 wvp, bvp, Pmat, Tmat, Smat, scq):
    BN = 1264
    grid = (NP // BN,)
    full = lambda a: pl.BlockSpec(a.shape, lambda i: (0,) * a.ndim)
    row = lambda w: pl.BlockSpec((BN, w), lambda i: (i, 0))
    return pl.pallas_call(
        _tables_body,
        grid=grid,
        in_specs=[row(C_S), row(9), row(3)] + [full(a) for a in
                  (wq, bq, wk, bk, wv, bv, wqp, bqp, wkp, bkp, wvp, bvp,
                   Pmat, Tmat, Smat, scq)],
        out_specs=[row(TBL_W), row(TBL_W), row(128), row(96), row(96),
                   row(48), row(48)],
        out_shape=[jax.ShapeDtypeStruct((NP, TBL_W), jnp.float32),
                   jax.ShapeDtypeStruct((NP, TBL_W), jnp.float32),
                   jax.ShapeDtypeStruct((NP, 128), jnp.float32),
                   jax.ShapeDtypeStruct((NP, 96), jnp.float32),
                   jax.ShapeDtypeStruct((NP, 96), jnp.float32),
                   jax.ShapeDtypeStruct((NP, 48), jnp.float32),
                   jax.ShapeDtypeStruct((NP, 48), jnp.float32)],
    )(s, r9, t, wq, bq, wk, bk, wv, bv, wqp, bqp, wkp, bkp, wvp, bvp,
      Pmat, Tmat, Smat, scq)


def _bs_body(z_ref, w_ref, b_ref, out_ref):
    out_ref[...] = jnp.dot(z_ref[...], w_ref[...],
                           preferred_element_type=jnp.float32) + b_ref[...]


def _make_bs(z, wb, bb):
    E = z.shape[0]
    BE = 1024
    assert E % BE == 0
    return pl.pallas_call(
        _bs_body,
        grid=(E // BE,),
        in_specs=[pl.BlockSpec((BE, C_Z), lambda i: (i, 0)),
                  pl.BlockSpec(wb.shape, lambda i: (0, 0)),
                  pl.BlockSpec(bb.shape, lambda i: (0,))],
        out_specs=pl.BlockSpec((BE, H), lambda i: (i, 0)),
        out_shape=jax.ShapeDtypeStruct((E, H), jnp.float32),
    )(z, wb * math.sqrt(1.0 / 3.0), bb * math.sqrt(1.0 / 3.0))


def _fin_body(s_ref, r_ref, t_ref, accv_ref, accvp_ref, uop_ref,
              E8h_ref, Wo_ref, Woi_ref, Won_ref, bo_ref,
              g_ref, b_ref, out_ref):
    accv = accv_ref[0] + accv_ref[1]
    den8 = accv[:, 128:136]
    recip = jnp.where(den8 > 0.0, 1.0 / jnp.where(den8 > 0.0, den8, 1.0), 0.0)
    E8h = E8h_ref[...]
    recip64 = jnp.dot(recip, E8h, preferred_element_type=jnp.float32,
                precision=jax.lax.Precision.HIGHEST)
    ow = accv[:, 0:128]
    o = jnp.concatenate(
        [ow[:, 16 * h:16 * (h + 1)] * recip[:, h:h + 1] for h in range(H)], axis=1)
    accvp = accvp_ref[0] + accvp_ref[1]
    t = t_ref[...]
    r = r_ref[...]
    op = [accvp[:, 64 * i:64 * (i + 1)] * recip64 - t[:, i:i + 1]
          for i in range(3)]
    orot = [r[:, 0 + i:1 + i] * op[0] + r[:, 3 + i:4 + i] * op[1]
            + r[:, 6 + i:7 + i] * op[2] for i in range(3)]
    opn = jnp.sqrt(orot[0] ** 2 + orot[1] ** 2 + orot[2] ** 2 + 1e-8)
    u = (jnp.dot(o, Wo_ref[...], preferred_element_type=jnp.float32,
                precision=jax.lax.Precision.HIGHEST)
         + bo_ref[...]
         + jnp.dot(orot[0], Woi_ref[0], preferred_element_type=jnp.float32,
                precision=jax.lax.Precision.HIGHEST)
         + jnp.dot(orot[1], Woi_ref[1], preferred_element_type=jnp.float32,
                precision=jax.lax.Precision.HIGHEST)
         + jnp.dot(orot[2], Woi_ref[2], preferred_element_type=jnp.float32,
                precision=jax.lax.Precision.HIGHEST)
         + jnp.dot(opn, Won_ref[...], preferred_element_type=jnp.float32,
                precision=jax.lax.Precision.HIGHEST))
    u += uop_ref[0] + uop_ref[1]
    x = s_ref[...] + u
    out_ref[...] = _ln_blk(x, g_ref[...], b_ref[...])


def _make_finalize(s, r9, t, accv, accvp, uop, E8h, Wo, Woi, Won, bo,
                   ln_g, ln_b):
    BN = 400
    full = lambda a: pl.BlockSpec(a.shape, lambda i: (0,) * a.ndim)
    return pl.pallas_call(
        _fin_body,
        grid=(N // BN,),
        in_specs=[pl.BlockSpec((BN, C_S), lambda i: (i, 0)),
                  pl.BlockSpec((BN, 9), lambda i: (i, 0)),
                  pl.BlockSpec((BN, 3), lambda i: (i, 0)),
                  pl.BlockSpec((2, BN, 144), lambda i: (0, i, 0)),
                  pl.BlockSpec((2, BN, 192), lambda i: (0, i, 0)),
                  pl.BlockSpec((2, BN, 128), lambda i: (0, i, 0)),
                  full(E8h), full(Wo), full(Woi), full(Won),
                  full(bo), full(ln_g), full(ln_b)],
        out_specs=pl.BlockSpec((BN, C_S), lambda i: (i, 0)),
        out_shape=jax.ShapeDtypeStruct((N, C_S), jnp.float32),
    )(s, r9, t, accv, accvp, uop, E8h, Wo, Woi, Won, bo, ln_g, ln_b)


def _node_fin_body(s_ref, r_ref, t_ref, nm_ref, w1, b1, w2, b2, w3, b3,
                   g_ref, be_ref, wbb, bbb, s_out, rn_out, tn_out):
    s = s_ref[...]
    x = jnp.maximum(jnp.dot(s, w1[...], preferred_element_type=jnp.float32,
                precision=jax.lax.Precision.HIGHEST) + b1[...], 0.0)
    x = jnp.maximum(jnp.dot(x, w2[...], preferred_element_type=jnp.float32,
                precision=jax.lax.Precision.HIGHEST) + b2[...], 0.0)
    x = jnp.dot(x, w3[...], preferred_element_type=jnp.float32,
                precision=jax.lax.Precision.HIGHEST) + b3[...]
    s3 = _ln_blk(s + x, g_ref[...], be_ref[...])
    s_out[...] = s3
    nm = nm_ref[...]
    upd = (jnp.dot(s3 * nm, wbb[...], preferred_element_type=jnp.float32,
                precision=jax.lax.Precision.HIGHEST)
           + bbb[...]) * nm
    u0 = upd[:, 0:1]; u1 = upd[:, 1:2]; u2 = upd[:, 2:3]
    n2 = 1.0 + u0 * u0 + u1 * u1 + u2 * u2
    inv = 1.0 / n2
    r00 = 1.0 - 2.0 * (u1 * u1 + u2 * u2) * inv
    r01 = 2.0 * (u0 * u1 - u2) * inv
    r02 = 2.0 * (u0 * u2 + u1) * inv
    r10 = 2.0 * (u0 * u1 + u2) * inv
    r11 = 1.0 - 2.0 * (u0 * u0 + u2 * u2) * inv
    r12 = 2.0 * (u1 * u2 - u0) * inv
    r20 = 2.0 * (u0 * u2 - u1) * inv
    r21 = 2.0 * (u1 * u2 + u0) * inv
    r22 = 1.0 - 2.0 * (u0 * u0 + u1 * u1) * inv
    rq = [[r00, r01, r02], [r10, r11, r12], [r20, r21, r22]]
    r = r_ref[...]
    t = t_ref[...]
    for i in range(3):
        for kk in range(3):
            rn_out[:, 3 * i + kk:3 * i + kk + 1] = (
                r[:, 3 * i + 0:3 * i + 1] * rq[0][kk]
                + r[:, 3 * i + 1:3 * i + 2] * rq[1][kk]
                + r[:, 3 * i + 2:3 * i + 3] * rq[2][kk])
        tn_out[:, i:i + 1] = (t[:, i:i + 1]
                              + r[:, 3 * i + 0:3 * i + 1] * upd[:, 3:4]
                              + r[:, 3 * i + 1:3 * i + 2] * upd[:, 4:5]
                              + r[:, 3 * i + 2:3 * i + 3] * upd[:, 5:6])


def _make_node_fin(s, r9, t, nm, p_nt, p_bb):
    BN = 400
    full = lambda a: pl.BlockSpec(a.shape, lambda i: (0,) * a.ndim)
    args = (p_nt['l1']['w'], p_nt['l1']['b'], p_nt['l2']['w'], p_nt['l2']['b'],
            p_nt['l3']['w'], p_nt['l3']['b'], p_nt['ln']['g'], p_nt['ln']['b'],
            p_bb['w'], p_bb['b'])
    return pl.pallas_call(
        _node_fin_body,
        grid=(N // BN,),
        in_specs=[pl.BlockSpec((BN, C_S), lambda i: (i, 0)),
                  pl.BlockSpec((BN, 9), lambda i: (i, 0)),
                  pl.BlockSpec((BN, 3), lambda i: (i, 0)),
                  pl.BlockSpec((BN, 1), lambda i: (i, 0))] +
                 [full(a) for a in args],
        out_specs=[pl.BlockSpec((BN, C_S), lambda i: (i, 0)),
                   pl.BlockSpec((BN, 9), lambda i: (i, 0)),
                   pl.BlockSpec((BN, 3), lambda i: (i, 0))],
        out_shape=[jax.ShapeDtypeStruct((N, C_S), jnp.float32),
                   jax.ShapeDtypeStruct((N, 9), jnp.float32),
                   jax.ShapeDtypeStruct((N, 3), jnp.float32)],
    )(s, r9, t, nm, *args)


def _ne_body(s_ref, w1, b1, w2, b2, o1, o2):
    s = s_ref[...]
    o1[...] = jnp.dot(s, w1[...], preferred_element_type=jnp.float32,
                precision=jax.lax.Precision.HIGHEST) + b1[...]
    o2[...] = jnp.dot(s, w2[...], preferred_element_type=jnp.float32,
                precision=jax.lax.Precision.HIGHEST) + b2[...]


def _make_ne(s, p_et, p_set):
    BN = 400
    full = lambda a: pl.BlockSpec(a.shape, lambda i: (0,) * a.ndim)
    args = (p_et['init']['w'], p_et['init']['b'],
            p_set['init']['w'], p_set['init']['b'])
    return pl.pallas_call(
        _ne_body,
        grid=(N // BN,),
        in_specs=[pl.BlockSpec((BN, C_S), lambda i: (i, 0))] +
                 [full(a) for a in args],
        out_specs=[pl.BlockSpec((BN, 64), lambda i: (i, 0)),
                   pl.BlockSpec((BN, 64), lambda i: (i, 0))],
        out_shape=[jax.ShapeDtypeStruct((N, 64), jnp.float32),
                   jax.ShapeDtypeStruct((N, 64), jnp.float32)],
    )(s, *args)


def _et_body(z_ref, ns_ref, nd_ref, w1a, w1b, w1c, b1, w2, b2,
             wfa, wfb, wfc, wfx, bf, g_ref, be_ref, out_ref):
    z = z_ref[...]
    ns = ns_ref[...]
    nd = nd_ref[...]
    x1 = jnp.maximum(
        jnp.dot(z, w1a[...], preferred_element_type=jnp.float32)
        + jnp.dot(ns, w1b[...], preferred_element_type=jnp.float32)
        + jnp.dot(nd, w1c[...], preferred_element_type=jnp.float32)
        + b1[...], 0.0)
    x2 = jnp.maximum(
        jnp.dot(x1, w2[...], preferred_element_type=jnp.float32) + b2[...], 0.0)
    e = (jnp.dot(x2, wfx[...], preferred_element_type=jnp.float32)
         + jnp.dot(z, wfa[...], preferred_element_type=jnp.float32)
         + jnp.dot(ns, wfb[...], preferred_element_type=jnp.float32)
         + jnp.dot(nd, wfc[...], preferred_element_type=jnp.float32)
         + bf[...])
    out_ref[...] = _ln_blk(e, g_ref[...], be_ref[...])


def _make_et(z, ns, nd, p):
    E = z.shape[0]
    BE = 1024
    assert E % BE == 0
    w1 = p['t1']['w']
    wf = p['final']['w']
    args = (w1[0:64], w1[64:128], w1[128:192], p['t1']['b'],
            p['t2']['w'], p['t2']['b'],
            wf[0:64], wf[64:128], wf[128:192], wf,
            p['final']['b'], p['ln']['g'], p['ln']['b'])
    full = lambda a: pl.BlockSpec(a.shape, lambda i: (0,) * a.ndim)
    return pl.pallas_call(
        _et_body,
        grid=(E // BE,),
        in_specs=[pl.BlockSpec((BE, 64), lambda i: (i, 0))] * 3 +
                 [full(a) for a in args],
        out_specs=pl.BlockSpec((BE, 64), lambda i: (i, 0)),
        out_shape=jax.ShapeDtypeStruct((E, 64), jnp.float32),
    )(z, ns, nd, *args)


def _sc_mesh():
    return plsc.VectorSubcoreMesh(core_axis_name="c", subcore_axis_name="s")


def _allsum(v, lane):
    for sh in (1, 2, 4, 8):
        v = v + jnp.take(v, lane ^ sh)
    return v


def _zero_acc(rows_ref, acc_ref, sid, width):
    zv = jnp.zeros((16,), jnp.float32)
    nr = rows_ref.shape[0]

    def zr(r, _):
        for wb in range(width // 16):
            rows_ref[r, pl.ds(wb * 16, 16)] = zv
        return 0
    lax.fori_loop(0, nr, zr, 0)
    r0 = sid * RPT
    nfull = RPT // nr

    def zc(i, _):
        pltpu.sync_copy(rows_ref, acc_ref.at[pl.ds(r0 + i * nr, nr)])
        return 0
    lax.fori_loop(0, nfull, zc, 0)
    rem = RPT - nfull * nr
    if rem:
        pltpu.sync_copy(rows_ref.at[pl.ds(0, rem)],
                        acc_ref.at[pl.ds(r0 + nfull * nr, rem)])


def _edge_split(Epad):
    Epc = Epad // 2
    nch = Epc // CH
    kmax = (nch + 15) // 16
    return Epc, nch, kmax


def _sc_pass_a(mq, mk, src, dst):
    Epad = src.shape[0]
    Epc, nch, kmax = _edge_split(Epad)

    @functools.partial(
        pl.kernel,
        out_type=jax.ShapeDtypeStruct((2, NP, 16), jnp.float32),
        mesh=_sc_mesh(),
        compiler_params=pltpu.CompilerParams(use_tc_tiling_on_sc=False),
        scratch_types=[
            pltpu.VMEM((CH,), jnp.int32), pltpu.VMEM((CH,), jnp.int32),
            pltpu.VMEM((CH, 48), jnp.float32), pltpu.VMEM((CH, 48), jnp.float32),
            pltpu.SemaphoreType.DMA, pltpu.SemaphoreType.DMA,
            pltpu.VMEM((CH,), jnp.int32), pltpu.VMEM((CH,), jnp.int32),
            pltpu.VMEM((CH, 48), jnp.float32), pltpu.VMEM((CH, 48), jnp.float32),
            pltpu.SemaphoreType.DMA, pltpu.SemaphoreType.DMA,
            pltpu.VMEM((CH, 16), jnp.float32),
            pltpu.VMEM_SHARED((NP, 16), jnp.float32),
        ],
    )
    def kern(mq_h, mk_h, src_h, dst_h, out_h,
             sidx0, didx0, mqr0, mkr0, semq0, semk0,
             sidx1, didx1, mqr1, mkr1, semq1, semk1, rows, acc):
        cid = lax.axis_index("c")
        sid = lax.axis_index("s")
        lane = lax.iota(jnp.int32, 16)
        _zero_acc(rows, acc, sid, 16)
        plsc.subcore_barrier()
        bufs = ((sidx0, didx0, mqr0, mkr0, semq0, semk0),
                (sidx1, didx1, mqr1, mkr1, semq1, semk1))

        def issue(ch, b):
            sidx, didx, mqr, mkr, semq, semk = b
            e0 = cid * Epc + ch * CH
            pltpu.sync_copy(src_h.at[pl.ds(e0, CH)], sidx)
            pltpu.sync_copy(dst_h.at[pl.ds(e0, CH)], didx)
            pltpu.async_copy(mq_h.at[didx], mqr, semq)
            pltpu.async_copy(mk_h.at[sidx], mkr, semk)

        def work(b):
            sidx, didx, mqr, mkr, semq, semk = b
            pltpu.make_async_copy(mq_h.at[didx], mqr, semq).wait()
            pltpu.make_async_copy(mk_h.at[sidx], mkr, semk).wait()

            def edge(r, _):
                d = (mqr[r, pl.ds(0, 16)] * mkr[r, pl.ds(0, 16)]
                     + mqr[r, pl.ds(16, 16)] * mkr[r, pl.ds(16, 16)])
                a0 = (_allsum(d, lane) + mqr[r, pl.ds(32, 16)]
                      + mkr[r, pl.ds(32, 16)])
                rows[r, :] = jnp.exp(a0)
                return 0
            lax.fori_loop(0, CH, edge, 0)
            pltpu.sync_copy(rows, acc.at[didx], add=True)

        @pl.when(sid < nch)
        def _():
            issue(sid, bufs[0])

        def two(j, _):
            for par in (0, 1):
                k = 2 * j + par
                ch = sid + k * 16

                @pl.when(ch < nch)
                def _(ch=ch, par=par):
                    @pl.when(ch + 16 < nch)
                    def _():
                        issue(ch + 16, bufs[1 - par])
                    work(bufs[par])
            return 0
        lax.fori_loop(0, (kmax + 1) // 2, two, 0)
        plsc.subcore_barrier()
        pltpu.sync_copy(acc.at[pl.ds(sid * RPT, RPT)],
                        out_h.at[cid, pl.ds(sid * RPT, RPT)])
    return kern(mq, mk, src, dst)


def _mprep_body(u_ref, out_ref):
    u = u_ref[0, :, 0:1] + u_ref[1, :, 0:1]
    m = jnp.where(u > 1e-35, 16.0 * jnp.log(jnp.maximum(u, 1e-35)), 0.0)
    out_ref[...] = jnp.broadcast_to(m, (u.shape[0], 16))


def _make_mtab(U):
    BN = 1264
    return pl.pallas_call(
        _mprep_body,
        grid=(NP // BN,),
        in_specs=[pl.BlockSpec((2, BN, 16), lambda i: (0, i, 0))],
        out_specs=pl.BlockSpec((BN, 16), lambda i: (i, 0)),
        out_shape=jax.ShapeDtypeStruct((NP, 16), jnp.float32),
    )(U)


def _sc_scores(qtab, ktab, mtab, bsf, src, dst):
    Epad = src.shape[0]
    C = CH // 2
    Epc = Epad // 2
    nch = Epc // C
    kmax = (nch + 15) // 16

    @functools.partial(
        pl.kernel,
        out_type=jax.ShapeDtypeStruct((Epad * 8,), jnp.float32),
        mesh=_sc_mesh(),
        compiler_params=pltpu.CompilerParams(use_tc_tiling_on_sc=False),
        scratch_types=[
            pltpu.VMEM((C,), jnp.int32), pltpu.VMEM((C,), jnp.int32),
            pltpu.VMEM((C, TBL_W), jnp.float32),
            pltpu.VMEM((C, TBL_W), jnp.float32),
            pltpu.VMEM((C, 16), jnp.float32),
            pltpu.VMEM((C * 8,), jnp.float32),
            pltpu.SemaphoreType.DMA, pltpu.SemaphoreType.DMA,
            pltpu.SemaphoreType.DMA,
            pltpu.VMEM((C,), jnp.int32), pltpu.VMEM((C,), jnp.int32),
            pltpu.VMEM((C, TBL_W), jnp.float32),
            pltpu.VMEM((C, TBL_W), jnp.float32),
            pltpu.VMEM((C, 16), jnp.float32),
            pltpu.VMEM((C * 8,), jnp.float32),
            pltpu.SemaphoreType.DMA, pltpu.SemaphoreType.DMA,
            pltpu.SemaphoreType.DMA,
            pltpu.VMEM((C * 8,), jnp.float32),
        ],
    )
    def kern(q_h, k_h, m_h, bs_h, src_h, dst_h, ea_h,
             sidx0, didx0, qr0, kr0, mr0, bsb0, semq0, semk0, semm0,
             sidx1, didx1, qr1, kr1, mr1, bsb1, semq1, semk1, semm1, ab):
        cid = lax.axis_index("c")
        sid = lax.axis_index("s")
        lane = lax.iota(jnp.int32, 16)
        msk = lane < 8
        bufs = ((sidx0, didx0, qr0, kr0, mr0, bsb0, semq0, semk0, semm0),
                (sidx1, didx1, qr1, kr1, mr1, bsb1, semq1, semk1, semm1))

        def issue(ch, b):
            sidx, didx, qr, kr, mr, bsb, semq, semk, semm = b
            e0 = cid * Epc + ch * C
            pltpu.sync_copy(src_h.at[pl.ds(e0, C)], sidx)
            pltpu.sync_copy(dst_h.at[pl.ds(e0, C)], didx)
            pltpu.sync_copy(bs_h.at[pl.ds(e0 * 8, C * 8)], bsb)
            pltpu.async_copy(q_h.at[didx], qr, semq)
            pltpu.async_copy(k_h.at[sidx], kr, semk)
            pltpu.async_copy(m_h.at[didx], mr, semm)

        def work(ch, b):
            sidx, didx, qr, kr, mr, bsb, semq, semk, semm = b
            e0 = cid * Epc + ch * C
            pltpu.make_async_copy(q_h.at[didx], qr, semq).wait()
            pltpu.make_async_copy(k_h.at[sidx], kr, semk).wait()
            pltpu.make_async_copy(m_h.at[didx], mr, semm).wait()

            def pair(u, _):
                av = jnp.zeros((16,), jnp.float32)
                for l in range(16):
                    rr = 2 * u + (l // 8)
                    off = (l % 8) * 16
                    d = (qr[rr, pl.ds(off, 16)] * kr[rr, pl.ds(off, 16)]
                         + qr[rr, pl.ds(off + 128, 16)]
                         * kr[rr, pl.ds(off + 128, 16)])
                    av = jnp.where(lane == l, _allsum(d, lane), av)
                nq = jnp.where(msk, qr[2 * u, pl.ds(256, 16)],
                               qr[2 * u + 1, pl.ds(256, 16)])
                nk = jnp.where(msk, kr[2 * u, pl.ds(256, 16)],
                               kr[2 * u + 1, pl.ds(256, 16)])
                mv = jnp.where(msk, mr[2 * u, :], mr[2 * u + 1, :])
                ab[pl.ds(u * 16, 16)] = jnp.exp(
                    av + nq + nk - mv + bsb[pl.ds(u * 16, 16)])
                return 0
            lax.fori_loop(0, C // 2, pair, 0)
            pltpu.sync_copy(ab, ea_h.at[pl.ds(e0 * 8, C * 8)])

        @pl.when(sid < nch)
        def _():
            issue(sid, bufs[0])

        def two(j, _):
            for par in (0, 1):
                k = 2 * j + par
                ch = sid + k * 16

                @pl.when(ch < nch)
                def _(ch=ch, par=par):
                    @pl.when(ch + 16 < nch)
                    def _():
                        issue(ch + 16, bufs[1 - par])
                    work(ch, bufs[par])
            return 0
        lax.fori_loop(0, (kmax + 1) // 2, two, 0)
    return kern(qtab, ktab, mtab, bsf, src, dst)


def _sc_accumulate(tab_or_z, eaf, src, dst, mode, h0=0, hp=()):
    Epad = src.shape[0]
    C = 64 if mode == 'v' else CH
    Epc = Epad // 2
    nch = Epc // C
    kmax = (nch + 15) // 16
    width = {'v': 144, 'vp': 96, 'z': 128}[mode]
    in_w = {'v': 128, 'vp': 96, 'z': 64}[mode]

    @functools.partial(
        pl.kernel,
        out_type=jax.ShapeDtypeStruct((2, NP, width), jnp.float32),
        mesh=_sc_mesh(),
        compiler_params=pltpu.CompilerParams(use_tc_tiling_on_sc=False),
        scratch_types=[
            pltpu.VMEM((C,), jnp.int32), pltpu.VMEM((C,), jnp.int32),
            pltpu.VMEM((C, in_w), jnp.float32),
            pltpu.VMEM((C * 8 + 16,), jnp.float32),
            pltpu.SemaphoreType.DMA,
            pltpu.VMEM((C,), jnp.int32), pltpu.VMEM((C,), jnp.int32),
            pltpu.VMEM((C, in_w), jnp.float32),
            pltpu.VMEM((C * 8 + 16,), jnp.float32),
            pltpu.SemaphoreType.DMA,
            pltpu.VMEM((C, width), jnp.float32),
            pltpu.VMEM_SHARED((NP, width), jnp.float32),
        ],
    )
    def kern(t_h, ea_h, src_h, dst_h, out_h, sidx0, didx0, tr0, eab0, sem0,
             sidx1, didx1, tr1, eab1, sem1, rows, acc):
        cid = lax.axis_index("c")
        sid = lax.axis_index("s")
        msk = lax.iota(jnp.int32, 16) < 8
        _zero_acc(rows, acc, sid, width)
        eab0[pl.ds(C * 8, 16)] = jnp.zeros((16,), jnp.float32)
        eab1[pl.ds(C * 8, 16)] = jnp.zeros((16,), jnp.float32)
        plsc.subcore_barrier()
        bufs = ((sidx0, didx0, tr0, eab0, sem0),
                (sidx1, didx1, tr1, eab1, sem1))

        def issue(ch, b):
            sidx, didx, tr, eab, sem = b
            e0 = cid * Epc + ch * C
            pltpu.sync_copy(src_h.at[pl.ds(e0, C)], sidx)
            pltpu.sync_copy(dst_h.at[pl.ds(e0, C)], didx)
            pltpu.sync_copy(ea_h.at[pl.ds(e0 * 8, C * 8)],
                            eab.at[pl.ds(0, C * 8)])
            if mode == 'z':
                pltpu.async_copy(t_h.at[pl.ds(e0, C)], tr, sem)
            else:
                pltpu.async_copy(t_h.at[sidx], tr, sem)

        def work(b):
            sidx, didx, tr, eab, sem = b
            if mode == 'z':
                pltpu.make_async_copy(t_h.at[pl.ds(0, C)], tr, sem).wait()
            else:
                pltpu.make_async_copy(t_h.at[sidx], tr, sem).wait()

            def edge(r, _):
                ev = eab[pl.ds(r * 8, 16)]
                if mode == 'v':
                    rows[r, pl.ds(128, 16)] = ev
                    for h in range(8):
                        w = jnp.full((16,), ev[h], jnp.float32)
                        rows[r, pl.ds(h * 16, 16)] = (
                            w * tr[r, pl.ds(h * 16, 16)])
                elif mode == 'vp':
                    for cb in range(6):
                        h = hp[cb]
                        w = jnp.where(
                            msk,
                            jnp.full((16,), ev[h], jnp.float32),
                            jnp.full((16,), ev[h + 1], jnp.float32))
                        rows[r, pl.ds(cb * 16, 16)] = (
                            w * tr[r, pl.ds(cb * 16, 16)])
                else:
                    for j in range(2):
                        w = jnp.full((16,), ev[h0 + j], jnp.float32)
                        for cb in range(4):
                            rows[r, pl.ds(j * 64 + cb * 16, 16)] = (
                                w * tr[r, pl.ds(cb * 16, 16)])
                return 0
            lax.fori_loop(0, C, edge, 0)
            pltpu.sync_copy(rows, acc.at[didx], add=True)

        @pl.when(sid < nch)
        def _():
            issue(sid, bufs[0])

        def two(j, _):
            for par in (0, 1):
                k = 2 * j + par
                ch = sid + k * 16

                @pl.when(ch < nch)
                def _(ch=ch, par=par):
                    @pl.when(ch + 16 < nch)
                    def _():
                        issue(ch + 16, bufs[1 - par])
                    work(bufs[par])
            return 0
        lax.fori_loop(0, (kmax + 1) // 2, two, 0)
        plsc.subcore_barrier()
        pltpu.sync_copy(acc.at[pl.ds(sid * RPT, RPT)],
                        out_h.at[cid, pl.ds(sid * RPT, RPT)])
    return kern(tab_or_z, eaf, src, dst)


def _dentab_body(accv_ref, out_ref):
    den8 = accv_ref[0][:, 128:136] + accv_ref[1][:, 128:136]
    inv = jnp.where(den8 > 0.0, 1.0 / jnp.where(den8 > 0.0, den8, 1.0), 0.0)
    out_ref[...] = jnp.concatenate([inv, jnp.zeros_like(inv)], axis=1)


def _make_dentab(accv):
    BN = 1264
    return pl.pallas_call(
        _dentab_body,
        grid=(NP // BN,),
        in_specs=[pl.BlockSpec((2, BN, 144), lambda i: (0, i, 0))],
        out_specs=pl.BlockSpec((BN, 16), lambda i: (i, 0)),
        out_shape=jax.ShapeDtypeStruct((NP, 16), jnp.float32),
    )(accv)


def _yz_body(z_ref, ea_ref, inv_ref, wog_ref, out_ref):
    z = z_ref[...]
    attn = ea_ref[...] * inv_ref[:, 0:8]
    wog = wog_ref[...]
    acc = attn[:, 0:1] * jnp.dot(z, wog[0:64],
                                 preferred_element_type=jnp.float32)
    for h in range(1, H):
        acc += attn[:, h:h + 1] * jnp.dot(z, wog[64 * h:64 * (h + 1)],
                                          preferred_element_type=jnp.float32)
    out_ref[...] = acc


def _make_yz(z, ea, invd, Wog):
    E = z.shape[0]
    BE = 1024
    full = lambda a: pl.BlockSpec(a.shape, lambda i: (0,) * a.ndim)
    return pl.pallas_call(
        _yz_body,
        grid=(E // BE,),
        in_specs=[pl.BlockSpec((BE, C_Z), lambda i: (i, 0)),
                  pl.BlockSpec((BE, H), lambda i: (i, 0)),
                  pl.BlockSpec((BE, 16), lambda i: (i, 0)),
                  full(Wog)],
        out_specs=pl.BlockSpec((BE, 128), lambda i: (i, 0)),
        out_shape=jax.ShapeDtypeStruct((E, 128), jnp.float32),
    )(z, ea, invd, Wog)


def _sc_gather1(tab, idx):
    Epad = idx.shape[0]
    Epc, nch, kmax = _edge_split(Epad)
    w = tab.shape[1]

    @functools.partial(
        pl.kernel,
        out_type=jax.ShapeDtypeStruct((Epad, w), jnp.float32),
        mesh=_sc_mesh(),
        compiler_params=pltpu.CompilerParams(use_tc_tiling_on_sc=False),
        scratch_types=[
            pltpu.VMEM((CH,), jnp.int32),
            pltpu.VMEM((CH, w), jnp.float32),
            pltpu.SemaphoreType.DMA,
        ],
    )
    def kern(tab_h, idx_h, out_h, didx, buf, sem1):
        cid = lax.axis_index("c")
        sid = lax.axis_index("s")

        def chunk(k, _):
            ch = sid + k * 16

            @pl.when(ch < nch)
            def _():
                e0 = cid * Epc + ch * CH
                pltpu.sync_copy(idx_h.at[pl.ds(e0, CH)], didx)
                pltpu.async_copy(tab_h.at[didx], buf, sem1).wait()
                pltpu.sync_copy(buf, out_h.at[pl.ds(e0, CH)])
            return 0
        lax.fori_loop(0, kmax, chunk, 0)
    return kern(tab, idx)


def _sc_scatter_rows(rows_tab, src, dst):
    Epad = src.shape[0]
    Epc, nch, kmax = _edge_split(Epad)

    @functools.partial(
        pl.kernel,
        out_type=jax.ShapeDtypeStruct((2, NP, 128), jnp.float32),
        mesh=_sc_mesh(),
        compiler_params=pltpu.CompilerParams(use_tc_tiling_on_sc=False),
        scratch_types=[
            pltpu.VMEM((CH,), jnp.int32),
            pltpu.VMEM((CH, 128), jnp.float32),
            pltpu.SemaphoreType.DMA,
            pltpu.VMEM_SHARED((NP, 128), jnp.float32),
        ],
    )
    def kern(rows_h, dst_h, out_h, didx, buf, sem1, acc):
        cid = lax.axis_index("c")
        sid = lax.axis_index("s")
        _zero_acc(buf, acc, sid, 128)
        plsc.subcore_barrier()

        def chunk(k, _):
            ch = sid + k * 16

            @pl.when(ch < nch)
            def _():
                e0 = cid * Epc + ch * CH
                pltpu.sync_copy(dst_h.at[pl.ds(e0, CH)], didx)
                pltpu.async_copy(rows_h.at[pl.ds(e0, CH)], buf, sem1).wait()
                pltpu.sync_copy(buf, acc.at[didx], add=True)
            return 0
        lax.fori_loop(0, kmax, chunk, 0)
        plsc.subcore_barrier()
        pltpu.sync_copy(acc.at[pl.ds(sid * RPT, RPT)],
                        out_h.at[cid, pl.ds(sid * RPT, RPT)])
    return kern(rows_tab, dst)


def _sc_gather_ne(tab, src, dst):
    Epad = src.shape[0]
    Epc, nch, kmax = _edge_split(Epad)

    @functools.partial(
        pl.kernel,
        out_type=[jax.ShapeDtypeStruct((Epad, 64), jnp.float32),
                  jax.ShapeDtypeStruct((Epad, 64), jnp.float32)],
        mesh=_sc_mesh(),
        compiler_params=pltpu.CompilerParams(use_tc_tiling_on_sc=False),
        scratch_types=[
            pltpu.VMEM((CH,), jnp.int32), pltpu.VMEM((CH,), jnp.int32),
            pltpu.VMEM((CH, 64), jnp.float32), pltpu.VMEM((CH, 64), jnp.float32),
            pltpu.SemaphoreType.DMA, pltpu.SemaphoreType.DMA,
            pltpu.VMEM((CH,), jnp.int32), pltpu.VMEM((CH,), jnp.int32),
            pltpu.VMEM((CH, 64), jnp.float32), pltpu.VMEM((CH, 64), jnp.float32),
            pltpu.SemaphoreType.DMA, pltpu.SemaphoreType.DMA,
        ],
    )
    def kern(tab_h, src_h, dst_h, outs_h, outd_h,
             sidx0, didx0, b10, b20, sems0, semd0,
             sidx1, didx1, b11, b21, sems1, semd1):
        cid = lax.axis_index("c")
        sid = lax.axis_index("s")
        bufs = ((sidx0, didx0, b10, b20, sems0, semd0),
                (sidx1, didx1, b11, b21, sems1, semd1))

        def issue(ch, b):
            sidx, didx, b1, b2, sems, semd = b
            e0 = cid * Epc + ch * CH
            pltpu.sync_copy(src_h.at[pl.ds(e0, CH)], sidx)
            pltpu.sync_copy(dst_h.at[pl.ds(e0, CH)], didx)
            pltpu.async_copy(tab_h.at[sidx], b1, sems)
            pltpu.async_copy(tab_h.at[didx], b2, semd)

        def work(ch, b):
            sidx, didx, b1, b2, sems, semd = b
            e0 = cid * Epc + ch * CH
            pltpu.make_async_copy(tab_h.at[sidx], b1, sems).wait()
            pltpu.make_async_copy(tab_h.at[didx], b2, semd).wait()
            pltpu.sync_copy(b1, outs_h.at[pl.ds(e0, CH)])
            pltpu.sync_copy(b2, outd_h.at[pl.ds(e0, CH)])

        @pl.when(sid < nch)
        def _():
            issue(sid, bufs[0])

        def two(j, _):
            for par in (0, 1):
                k = 2 * j + par
                ch = sid + k * 16

                @pl.when(ch < nch)
                def _(ch=ch, par=par):
                    @pl.when(ch + 16 < nch)
                    def _():
                        issue(ch + 16, bufs[1 - par])
                    work(ch, bufs[par])
            return 0
        lax.fori_loop(0, (kmax + 1) // 2, two, 0)
    return kern(tab, src, dst)


def _perm_ipa_weights(p):
    wq, bq = p['wq']['w'], p['wq']['b']
    wkv, bkv = p['wkv']['w'], p['wkv']['b']
    kcols = np.array([h * 32 + c for h in range(H) for c in range(C_H)])
    vcols = kcols + C_H
    wk, bk = wkv[:, kcols], bkv[kcols]
    wv, bv = wkv[:, vcols], bkv[vcols]
    qp_cols = np.array([hp * 3 + j for j in range(3) for hp in range(H * PQK)])
    wqp, bqp = p['wqp']['w'][:, qp_cols], p['wqp']['b'][qp_cols]
    kp_cols = np.array([(h * 12 + pq) * 3 + j
                        for j in range(3) for h in range(H) for pq in range(PQK)])
    vp_cols = np.array([(h * 12 + 4 + pv) * 3 + j
                        for j in range(3) for h in range(H) for pv in range(PV)])
    wkp, bkp = p['wkvp']['w'][:, kp_cols], p['wkvp']['b'][kp_cols]
    wvp, bvp = p['wkvp']['w'][:, vp_cols], p['wkvp']['b'][vp_cols]
    Pm = np.zeros((3, 32, 128), np.float32)
    Tm = np.zeros((3, 128), np.float32)
    for h in range(H):
        for i in range(3):
            for pq in range(PQK):
                Pm[i, h * 4 + pq, h * 16 + i * 4 + pq] = 1.0
                Tm[i, h * 16 + i * 4 + pq] = 1.0
    Sm = np.zeros((128, 16), np.float32)
    for h in range(H):
        Sm[h * 16:(h + 1) * 16, h] = 1.0
        Sm[h * 16:(h + 1) * 16, 8 + h] = 1.0
    hw2 = 0.5 * jax.nn.softplus(p['hw']) * math.sqrt(1.0 / (3.0 * (PQK * 9.0 / 2.0)))
    scq = jnp.sqrt(2.0 * hw2)[:, None].repeat(16, 1).reshape(1, 128)
    wo, bo = p['wo']['w'], p['wo']['b']
    Wo = wo[0:128]
    Woi = jnp.stack([wo[128 + np.array([h * 24 + pv * 3 + i
                                        for h in range(H) for pv in range(PV)])]
                     for i in range(3)], 0)
    Won = wo[320:384]
    Wog = wo[384:896]
    E8h = np.zeros((8, 64), np.float32)
    for h in range(H):
        E8h[h, h * 8:(h + 1) * 8] = 1.0
    return dict(wq=wq, bq=bq, wk=wk, bk=bk, wv=wv, bv=bv, wqp=wqp, bqp=bqp,
                wkp=wkp, bkp=bkp, wvp=wvp, bvp=bvp,
                P=jnp.asarray(Pm), T=jnp.asarray(Tm), S=jnp.asarray(Sm),
                scq=scq, Wo=Wo, Woi=Woi, Won=Won, Wog=Wog, bo=bo,
                E8h=jnp.asarray(E8h))


def _pad_rows(a, rows):
    return jnp.pad(a, ((0, rows - a.shape[0]),) + ((0, 0),) * (a.ndim - 1))


def _run_ipa(s, sp, r9, r9p, t, tp, zpad, bsf, src, dst, p, ln):
    w = _perm_ipa_weights(p)
    qtab, ktab, vtab, vptab0, vptab1, mq, mk = _make_tables(
        sp, r9p, tp, w['wq'], w['bq'], w['wk'], w['bk'], w['wv'], w['bv'],
        w['wqp'], w['bqp'], w['wkp'], w['bkp'], w['wvp'], w['bvp'],
        w['P'], w['T'], w['S'], w['scq'])
    U = _sc_pass_a(mq, mk, src, dst)
    mtab = _make_mtab(U)
    eaf = _sc_scores(qtab, ktab, mtab, bsf, src, dst)
    accv = _sc_accumulate(vtab, eaf, src, dst, 'v')
    accvp = jnp.concatenate(
        [_sc_accumulate(vptab0, eaf, src, dst, 'vp', hp=(0, 2, 4, 6, 0, 2)),
         _sc_accumulate(vptab1, eaf, src, dst, 'vp', hp=(4, 6, 0, 2, 4, 6))],
        axis=-1)
    dentab = _make_dentab(accv)
    invd = _sc_gather1(dentab, dst)
    yz = _make_yz(zpad, eaf.reshape(-1, 8), invd, w['Wog'])
    uop = _sc_scatter_rows(yz, src, dst)
    return _make_finalize(s, r9, t, accv, accvp, uop, w['E8h'], w['Wo'],
                          w['Woi'], w['Won'], w['bo'],
                          ln['g'], ln['b'])


def kernel(node_features, rots, trans, edge_features, edge_index,
           seq_edge_features, seq_edge_index, res_mask, noising_mask, params):
    r9 = rots.reshape(N, 9)
    t = trans
    r9p = _pad_rows(r9, NP)
    tp = _pad_rows(t, NP)

    def prep(z, ei):
        E0 = ei.shape[1]
        Epad = -(-E0 // 1024) * 1024
        srcp = jnp.pad(ei[0], (0, Epad - E0))
        dstp = jnp.pad(ei[1], (0, Epad - E0), constant_values=N)
        return jnp.pad(z, ((0, Epad - E0), (0, 0))), srcp, dstp, E0

    zp, src, dst, E = prep(edge_features, edge_index)
    szp, ssrc, sdst, ES = prep(seq_edge_features, seq_edge_index)
    bsf = _make_bs(zp, params['ipa_sp']['wb']['w'],
                   params['ipa_sp']['wb']['b']).reshape(-1)
    sbsf = _make_bs(szp, params['ipa_sq']['wb']['w'],
                    params['ipa_sq']['wb']['b']).reshape(-1)
    s0p = _pad_rows(node_features, NP)
    s1 = _run_ipa(node_features, s0p, r9, r9p, t, tp, zp, bsf, src, dst,
                  params['ipa_sp'], params['ln1'])
    s2 = _run_ipa(s1, _pad_rows(s1, NP), r9, r9p, t, tp, szp, sbsf,
                  ssrc, sdst, params['ipa_sq'], params['ln2'])
    s3, rn9, tn = _make_node_fin(s2, r9, t, noising_mask[:, None],
                                 params['nt'], params['bb'])
    ne_et, ne_set = _make_ne(s3, params['et'], params['set'])
    nsrc, ndst = _sc_gather_ne(_pad_rows(ne_et, NP), src, dst)
    e = _make_et(zp, nsrc, ndst, params['et'])[:E]
    snsrc, sndst = _sc_gather_ne(_pad_rows(ne_set, NP), ssrc, sdst)
    se = _make_et(szp, snsrc, sndst, params['set'])[:ES]
    return s3, rn9.reshape(N, 3, 3), tn, e, se
